# baseline (speedup 1.0000x reference)
import functools

import jax
import jax.numpy as jnp
import numpy as np
from jax import lax
from jax.experimental import pallas as pl
from jax.experimental.pallas import tpu as pltpu
from jax.experimental.pallas import tpu_sc as plsc

N_NODES = 10000
N_EDGES = 320000
HIDDEN = 128
N_GRAPHS = 64

NC = 2
NS = 16
NW = NC * NS
CHUNK = 64
IBLK = 16
NIB = 10
RING = 4
CH = NIB * IBLK
EPW = CH * CHUNK
NP = 10240
ROWS_PER_TILE = NP // NS


def _make_sc_agg(n_rows: int):
    mesh = plsc.VectorSubcoreMesh(core_axis_name="c", subcore_axis_name="s")

    @functools.partial(
        pl.kernel,
        mesh=mesh,
        out_type=jax.ShapeDtypeStruct((NC, NP, HIDDEN), jnp.float32),
        scratch_types=[
            pltpu.VMEM((IBLK, CHUNK), jnp.int32),
            pltpu.VMEM((IBLK, CHUNK), jnp.int32),
            pltpu.VMEM((IBLK, CHUNK), jnp.int32),
            pltpu.VMEM((IBLK, CHUNK), jnp.int32),
        ] + [pltpu.VMEM((CHUNK, HIDDEN), jnp.float32) for _ in range(RING)]
          + [pltpu.VMEM_SHARED((NP, HIDDEN), jnp.float32)]
          + [pltpu.SemaphoreType.DMA for _ in range(2 + RING)],
    )
    def k(x_hbm, src_hbm, dst_hbm, run_hbm, out_hbm, sib0, sib1, dib0, dib1, *rest):
        sib = (sib0, sib1)
        dib = (dib0, dib1)
        rows = rest[:RING]
        acc = rest[RING]
        semi = rest[RING + 1: RING + 3]
        semr = rest[RING + 3:]
        c = lax.axis_index("c")
        s = lax.axis_index("s")
        w = c * NS + s

        def stage_idx(bi, p, runoff=False):
            if runoff:
                pltpu.async_copy(run_hbm.at[w], sib[p], semi[p])
                pltpu.async_copy(run_hbm.at[w], dib[p], semi[p])
            else:
                pltpu.async_copy(src_hbm.at[w, bi], sib[p], semi[p])
                pltpu.async_copy(dst_hbm.at[w, bi], dib[p], semi[p])

        def wait_idx(bi, p):
            pltpu.make_async_copy(src_hbm.at[w, 0], sib[p], semi[p]).wait()
            pltpu.make_async_copy(dst_hbm.at[w, 0], dib[p], semi[p]).wait()

        stage_idx(0, 0)

        def zrow(i, carry):
            for j in range(HIDDEN // 16):
                rows[0][i, pl.ds(j * 16, 16)] = jnp.zeros((16,), jnp.float32)
            return carry

        lax.fori_loop(0, CHUNK, zrow, 0)
        for kk in range(ROWS_PER_TILE // CHUNK):
            pltpu.sync_copy(rows[0], acc.at[pl.ds(s * ROWS_PER_TILE + kk * CHUNK, CHUNK)])

        wait_idx(0, 0)
        plsc.subcore_barrier()

        for b in range(RING):
            pltpu.async_copy(x_hbm.at[sib[0].at[b]], rows[b], semr[b])

        def process_block(i, p, next_is_runoff=False):
            stage_idx(i + 1, p ^ 1, runoff=next_is_runoff)
            for q in range(IBLK):
                b = q % RING
                if q == IBLK - RING:
                    wait_idx(i + 1, p ^ 1)
                pltpu.make_async_copy(x_hbm.at[sib[p].at[q]], rows[b], semr[b]).wait()
                pltpu.sync_copy(rows[b], acc.at[dib[p].at[q]], add=True)
                if q < IBLK - RING:
                    pltpu.async_copy(x_hbm.at[sib[p].at[q + RING]], rows[b], semr[b])
                else:
                    pltpu.async_copy(x_hbm.at[sib[p ^ 1].at[q + RING - IBLK]], rows[b], semr[b])

        def body(i2, carry):
            process_block(2 * i2, 0)
            process_block(2 * i2 + 1, 1)
            return carry

        lax.fori_loop(0, NIB // 2 - 1, body, 0)
        process_block(NIB - 2, 0)
        process_block(NIB - 1, 1, next_is_runoff=True)
        for b in range(RING):
            pltpu.make_async_copy(x_hbm.at[sib[0].at[b]], rows[b], semr[b]).wait()
        plsc.subcore_barrier()

        pltpu.sync_copy(
            acc.at[pl.ds(s * ROWS_PER_TILE, ROWS_PER_TILE)],
            out_hbm.at[c, pl.ds(s * ROWS_PER_TILE, ROWS_PER_TILE)],
        )

    return k


_DE_COLS = 32768
_DE_ROWS = _DE_COLS // 128
_TOT = NW * NIB * IBLK * CHUNK
_DE_G = _TOT // _DE_COLS


def _dein_body(e0_ref, e1_ref, src_ref, dst_ref):
    g = pl.program_id(0)

    @pl.when(g < _DE_G - 1)
    def _():
        src_ref[...] = e0_ref[0, 0].reshape(_DE_ROWS, 128)
        dst_ref[...] = e1_ref[0, 0].reshape(_DE_ROWS, 128)

    @pl.when(g == _DE_G - 1)
    def _():
        fp = (g * _DE_COLS
              + lax.broadcasted_iota(jnp.int32, (_DE_ROWS, 128), 0) * 128
              + lax.broadcasted_iota(jnp.int32, (_DE_ROWS, 128), 1))
        m = fp < N_EDGES
        q = fp - N_EDGES
        src_ref[...] = jnp.where(m, e0_ref[0, 0].reshape(_DE_ROWS, 128), q)
        dst_ref[...] = jnp.where(m, e1_ref[0, 0].reshape(_DE_ROWS, 128),
                                 N_NODES + (q >> 5))


def _dein(edge_index):
    e3 = edge_index.reshape(2, 1, N_EDGES)
    spec0 = pl.BlockSpec((1, 1, _DE_COLS), lambda g: (0, 0, g))
    spec1 = pl.BlockSpec((1, 1, _DE_COLS), lambda g: (1, 0, g))
    out = pl.BlockSpec((_DE_ROWS, 128), lambda g: (g, 0))
    src2d, dst2d = pl.pallas_call(
        _dein_body,
        grid=(_DE_G,),
        in_specs=[spec0, spec1],
        out_specs=[out, out],
        out_shape=[jax.ShapeDtypeStruct((_TOT // 128, 128), jnp.int32)] * 2,
    )(e3, e3)
    shape4 = (NW, NIB, IBLK, CHUNK)
    return src2d.reshape(shape4), dst2d.reshape(shape4)


_RUNOFF = ((np.arange(NW * IBLK * CHUNK, dtype=np.int32) * 613) % N_NODES
           ).reshape(NW, IBLK, CHUNK)


_INV_SQRT2 = 0.7071067811865476


def _gelu(z):
    return 0.5 * z * (1.0 + lax.erf(z * _INV_SQRT2))


def _mlp1_body(x_ref, a0_ref, a1_ref, wa_ref, ba_ref, wb_ref, bb_ref, o_ref):
    t = x_ref[...] + a0_ref[0] + a1_ref[0]
    z = jnp.dot(t, wa_ref[...], preferred_element_type=jnp.float32) + ba_ref[...]
    z = _gelu(z)
    z = jnp.dot(z, wb_ref[...], preferred_element_type=jnp.float32) + bb_ref[...]
    o_ref[...] = _gelu(z)


def _mlp2_body(x_ref, a0_ref, a1_ref, wa_ref, ba_ref, wb_ref, bb_ref,
               wfc_ref, bfc_ref, b_ref, o_ref):
    t = x_ref[...] + a0_ref[0] + a1_ref[0]
    z = jnp.dot(t, wa_ref[...], preferred_element_type=jnp.float32) + ba_ref[...]
    z = _gelu(z)
    z = jnp.dot(z, wb_ref[...], preferred_element_type=jnp.float32) + bb_ref[...]
    z = _gelu(z)
    v = jnp.dot(z, wfc_ref[...], preferred_element_type=jnp.float32) + bfc_ref[...]
    bb = b_ref[0, 0, :]
    onehot = (bb[None, :] == lax.broadcasted_iota(jnp.int32, (N_GRAPHS, bb.shape[0]), 0)
              ).astype(jnp.float32)
    pooled = jnp.dot(onehot, v, preferred_element_type=jnp.float32)

    @pl.when(pl.program_id(0) == 0)
    def _():
        o_ref[...] = jnp.zeros_like(o_ref)

    o_ref[...] += pooled


_BLK = 400
_GRID = N_NODES // _BLK


_ROW = pl.BlockSpec((_BLK, HIDDEN), lambda i: (i, 0))
_AGG0 = pl.BlockSpec((1, _BLK, HIDDEN), lambda i: (0, i, 0))
_AGG1 = pl.BlockSpec((1, _BLK, HIDDEN), lambda i: (1, i, 0))
_FULL = pl.BlockSpec((HIDDEN, HIDDEN), lambda i: (0, 0))
_BIAS = pl.BlockSpec((1, HIDDEN), lambda i: (0, 0))


def _mlp1(x, agg, wa, ba, wb, bb):
    return pl.pallas_call(
        _mlp1_body,
        grid=(_GRID,),
        in_specs=[_ROW, _AGG0, _AGG1, _FULL, _BIAS, _FULL, _BIAS],
        out_specs=_ROW,
        out_shape=jax.ShapeDtypeStruct((N_NODES, HIDDEN), jnp.float32),
    )(x, agg, agg, wa, ba.reshape(1, HIDDEN), wb, bb.reshape(1, HIDDEN))


def _mlp2(h1, agg, wa, ba, wb, bb, wfc, bfc, batch):
    return pl.pallas_call(
        _mlp2_body,
        grid=(_GRID,),
        in_specs=[_ROW, _AGG0, _AGG1, _FULL, _BIAS, _FULL, _BIAS,
                  pl.BlockSpec((HIDDEN, 1), lambda i: (0, 0)),
                  pl.BlockSpec((1, 1), lambda i: (0, 0)),
                  pl.BlockSpec((1, 1, _BLK), lambda i: (i, 0, 0))],
        out_specs=pl.BlockSpec((N_GRAPHS, 1), lambda i: (0, 0)),
        out_shape=jax.ShapeDtypeStruct((N_GRAPHS, 1), jnp.float32),
    )(h1, agg, agg, wa, ba.reshape(1, HIDDEN), wb, bb.reshape(1, HIDDEN),
      wfc, bfc.reshape(1, 1), batch.reshape(_GRID, 1, _BLK))


def kernel(x, edge_index, batch, W1a, b1a, W1b, b1b, W2a, b2a, W2b, b2b, Wfc, bfc):
    src_p, dst_p = _dein(edge_index)
    runoff = jnp.asarray(_RUNOFF)
    agg1 = _make_sc_agg(N_NODES)(x, src_p, dst_p, runoff)
    h1 = _mlp1(x, agg1, W1a, b1a, W1b, b1b)
    agg2 = _make_sc_agg(N_NODES)(h1, src_p, dst_p, runoff)
    out = _mlp2(h1, agg2, W2a, b2a, W2b, b2b, Wfc, bfc, batch)
    return out


# speedup vs baseline: 1.0712x; 1.0712x over previous
import functools

import jax
import jax.numpy as jnp
import numpy as np
from jax import lax
from jax.experimental import pallas as pl
from jax.experimental.pallas import tpu as pltpu
from jax.experimental.pallas import tpu_sc as plsc

N_NODES = 10000
N_EDGES = 320000
HIDDEN = 128
N_GRAPHS = 64

NC = 2
NS = 16
NW = NC * NS
CHUNK = 64
IBLK = 16
NIB = 10
RING = 4
CH = NIB * IBLK
EPW = CH * CHUNK
NP = 10240
ROWS_PER_TILE = NP // NS


def _make_sc_agg(n_rows: int):
    mesh = plsc.VectorSubcoreMesh(core_axis_name="c", subcore_axis_name="s")

    @functools.partial(
        pl.kernel,
        mesh=mesh,
        out_type=jax.ShapeDtypeStruct((NC, NP, HIDDEN), jnp.float32),
        scratch_types=[
            pltpu.VMEM((IBLK, CHUNK), jnp.int32),
            pltpu.VMEM((IBLK, CHUNK), jnp.int32),
            pltpu.VMEM((IBLK, CHUNK), jnp.int32),
            pltpu.VMEM((IBLK, CHUNK), jnp.int32),
        ] + [pltpu.VMEM((CHUNK, HIDDEN), jnp.float32) for _ in range(RING)]
          + [pltpu.VMEM_SHARED((NP, HIDDEN), jnp.float32)]
          + [pltpu.SemaphoreType.DMA for _ in range(2 + RING)],
    )
    def k(x_hbm, src_hbm, dst_hbm, run_hbm, out_hbm, sib0, sib1, dib0, dib1, *rest):
        sib = (sib0, sib1)
        dib = (dib0, dib1)
        rows = rest[:RING]
        acc = rest[RING]
        semi = rest[RING + 1: RING + 3]
        semr = rest[RING + 3:]
        c = lax.axis_index("c")
        s = lax.axis_index("s")
        w = c * NS + s

        def stage_idx(bi, p, runoff=False):
            if runoff:
                pltpu.async_copy(run_hbm.at[w], sib[p], semi[p])
                pltpu.async_copy(run_hbm.at[w], dib[p], semi[p])
            else:
                pltpu.async_copy(src_hbm.at[w, bi], sib[p], semi[p])
                pltpu.async_copy(dst_hbm.at[w, bi], dib[p], semi[p])

        def wait_idx(bi, p):
            pltpu.make_async_copy(src_hbm.at[w, 0], sib[p], semi[p]).wait()
            pltpu.make_async_copy(dst_hbm.at[w, 0], dib[p], semi[p]).wait()

        stage_idx(0, 0)

        def zrow(i, carry):
            for j in range(HIDDEN // 16):
                rows[0][i, pl.ds(j * 16, 16)] = jnp.zeros((16,), jnp.float32)
            return carry

        lax.fori_loop(0, CHUNK, zrow, 0)
        for kk in range(ROWS_PER_TILE // CHUNK):
            pltpu.sync_copy(rows[0], acc.at[pl.ds(s * ROWS_PER_TILE + kk * CHUNK, CHUNK)])

        wait_idx(0, 0)
        plsc.subcore_barrier()

        for b in range(RING):
            pltpu.async_copy(x_hbm.at[sib[0].at[b]], rows[b], semr[b])

        def process_block(i, p, next_is_runoff=False):
            stage_idx(i + 1, p ^ 1, runoff=next_is_runoff)
            for q in range(IBLK):
                b = q % RING
                if q == IBLK - RING:
                    wait_idx(i + 1, p ^ 1)
                pltpu.make_async_copy(x_hbm.at[sib[p].at[q]], rows[b], semr[b]).wait()
                pltpu.sync_copy(rows[b], acc.at[dib[p].at[q]], add=True)
                if q < IBLK - RING:
                    pltpu.async_copy(x_hbm.at[sib[p].at[q + RING]], rows[b], semr[b])
                else:
                    pltpu.async_copy(x_hbm.at[sib[p ^ 1].at[q + RING - IBLK]], rows[b], semr[b])

        def body(i2, carry):
            process_block(2 * i2, 0)
            process_block(2 * i2 + 1, 1)
            return carry

        lax.fori_loop(0, NIB // 2 - 1, body, 0)
        process_block(NIB - 2, 0)
        process_block(NIB - 1, 1, next_is_runoff=True)
        for b in range(RING):
            pltpu.make_async_copy(x_hbm.at[sib[0].at[b]], rows[b], semr[b]).wait()
        plsc.subcore_barrier()

        pltpu.sync_copy(
            acc.at[pl.ds(s * ROWS_PER_TILE, ROWS_PER_TILE)],
            out_hbm.at[c, pl.ds(s * ROWS_PER_TILE, ROWS_PER_TILE)],
        )

    return k


_DE_COLS = 32768
_DE_ROWS = _DE_COLS // 128
_TOT = NW * NIB * IBLK * CHUNK
_DE_G = _TOT // _DE_COLS


def _dein_body(e0_ref, e1_ref, src_ref, dst_ref):
    g = pl.program_id(0)

    @pl.when(g < _DE_G - 1)
    def _():
        src_ref[...] = e0_ref[0, 0].reshape(_DE_ROWS, 128)
        dst_ref[...] = e1_ref[0, 0].reshape(_DE_ROWS, 128)

    @pl.when(g == _DE_G - 1)
    def _():
        fp = (g * _DE_COLS
              + lax.broadcasted_iota(jnp.int32, (_DE_ROWS, 128), 0) * 128
              + lax.broadcasted_iota(jnp.int32, (_DE_ROWS, 128), 1))
        m = fp < N_EDGES
        q = fp - N_EDGES
        src_ref[...] = jnp.where(m, e0_ref[0, 0].reshape(_DE_ROWS, 128), q)
        dst_ref[...] = jnp.where(m, e1_ref[0, 0].reshape(_DE_ROWS, 128),
                                 N_NODES + (q >> 5))


def _dein(edge_index):
    e3 = edge_index.reshape(2, 1, N_EDGES)
    spec0 = pl.BlockSpec((1, 1, _DE_COLS), lambda g: (0, 0, g))
    spec1 = pl.BlockSpec((1, 1, _DE_COLS), lambda g: (1, 0, g))
    out = pl.BlockSpec((_DE_ROWS, 128), lambda g: (g, 0))
    src2d, dst2d = pl.pallas_call(
        _dein_body,
        grid=(_DE_G,),
        in_specs=[spec0, spec1],
        out_specs=[out, out],
        out_shape=[jax.ShapeDtypeStruct((_TOT // 128, 128), jnp.int32)] * 2,
    )(e3, e3)
    shape4 = (NW, NIB, IBLK, CHUNK)
    return src2d.reshape(shape4), dst2d.reshape(shape4)


_RUNOFF = ((np.arange(NW * IBLK * CHUNK, dtype=np.int32) * 613) % N_NODES
           ).reshape(NW, IBLK, CHUNK)


_INV_SQRT2 = 0.7071067811865476


def _gelu(z):
    return 0.5 * z * (1.0 + lax.erf(z * _INV_SQRT2))


def _mlp1_body(x_ref, a0_ref, a1_ref, wa_ref, ba_ref, wb_ref, bb_ref, o_ref):
    t = x_ref[...] + a0_ref[0] + a1_ref[0]
    z = jnp.dot(t, wa_ref[...], preferred_element_type=jnp.float32) + ba_ref[...]
    z = _gelu(z)
    z = jnp.dot(z, wb_ref[...], preferred_element_type=jnp.float32) + bb_ref[...]
    o_ref[...] = _gelu(z)


def _mlp2_body(x_ref, a0_ref, a1_ref, wa_ref, ba_ref, wb_ref, bb_ref,
               wfc_ref, bfc_ref, b_ref, o_ref):
    t = x_ref[...] + a0_ref[0] + a1_ref[0]
    z = jnp.dot(t, wa_ref[...], preferred_element_type=jnp.float32) + ba_ref[...]
    z = _gelu(z)
    z = jnp.dot(z, wb_ref[...], preferred_element_type=jnp.float32) + bb_ref[...]
    z = _gelu(z)
    v = jnp.dot(z, wfc_ref[...], preferred_element_type=jnp.float32) + bfc_ref[...]
    bb = b_ref[0, 0, :]
    onehot = (bb[None, :] == lax.broadcasted_iota(jnp.int32, (N_GRAPHS, bb.shape[0]), 0)
              ).astype(jnp.float32)
    pooled = jnp.dot(onehot, v, preferred_element_type=jnp.float32)

    @pl.when(pl.program_id(0) == 0)
    def _():
        o_ref[...] = jnp.zeros_like(o_ref)

    o_ref[...] += pooled


_BLK = 1000
_GRID = N_NODES // _BLK


_ROW = pl.BlockSpec((_BLK, HIDDEN), lambda i: (i, 0))
_AGG0 = pl.BlockSpec((1, _BLK, HIDDEN), lambda i: (0, i, 0))
_AGG1 = pl.BlockSpec((1, _BLK, HIDDEN), lambda i: (1, i, 0))
_FULL = pl.BlockSpec((HIDDEN, HIDDEN), lambda i: (0, 0))
_BIAS = pl.BlockSpec((1, HIDDEN), lambda i: (0, 0))


def _mlp1(x, agg, wa, ba, wb, bb):
    return pl.pallas_call(
        _mlp1_body,
        grid=(_GRID,),
        in_specs=[_ROW, _AGG0, _AGG1, _FULL, _BIAS, _FULL, _BIAS],
        out_specs=_ROW,
        out_shape=jax.ShapeDtypeStruct((N_NODES, HIDDEN), jnp.float32),
    )(x, agg, agg, wa, ba.reshape(1, HIDDEN), wb, bb.reshape(1, HIDDEN))


def _mlp2(h1, agg, wa, ba, wb, bb, wfc, bfc, batch):
    return pl.pallas_call(
        _mlp2_body,
        grid=(_GRID,),
        in_specs=[_ROW, _AGG0, _AGG1, _FULL, _BIAS, _FULL, _BIAS,
                  pl.BlockSpec((HIDDEN, 1), lambda i: (0, 0)),
                  pl.BlockSpec((1, 1), lambda i: (0, 0)),
                  pl.BlockSpec((1, 1, _BLK), lambda i: (i, 0, 0))],
        out_specs=pl.BlockSpec((N_GRAPHS, 1), lambda i: (0, 0)),
        out_shape=jax.ShapeDtypeStruct((N_GRAPHS, 1), jnp.float32),
    )(h1, agg, agg, wa, ba.reshape(1, HIDDEN), wb, bb.reshape(1, HIDDEN),
      wfc, bfc.reshape(1, 1), batch.reshape(_GRID, 1, _BLK))


def kernel(x, edge_index, batch, W1a, b1a, W1b, b1b, W2a, b2a, W2b, b2b, Wfc, bfc):
    src_p, dst_p = _dein(edge_index)
    runoff = jnp.asarray(_RUNOFF)
    agg1 = _make_sc_agg(N_NODES)(x, src_p, dst_p, runoff)
    h1 = _mlp1(x, agg1, W1a, b1a, W1b, b1b)
    agg2 = _make_sc_agg(N_NODES)(h1, src_p, dst_p, runoff)
    out = _mlp2(h1, agg2, W2a, b2a, W2b, b2b, Wfc, bfc, batch)
    return out
